# baseline (speedup 1.0000x reference)
import functools

import jax
import jax.numpy as jnp
from jax import lax
from jax.experimental import pallas as pl
from jax.experimental.pallas import tpu as pltpu
from jax.experimental.pallas import tpu_sc as plsc

LANES = 16
CHUNK = 128
NBUF = 5


@functools.lru_cache(maxsize=None)
def _build(batch: int, seq: int, seq_pad: int, d: int, dpad: int,
           n_workers: int):
    b_per_w = batch // n_workers
    n_jt = b_per_w // CHUNK
    mesh = plsc.VectorSubcoreMesh(core_axis_name="c", subcore_axis_name="s")

    @functools.partial(
        pl.kernel,
        mesh=mesh,
        out_type=jax.ShapeDtypeStruct((seq, d, batch), jnp.float32),
        scratch_types=[
            pltpu.VMEM((seq_pad, CHUNK), jnp.int32),
            pltpu.VMEM((NBUF, CHUNK, dpad), jnp.float32),
            pltpu.VMEM((d, CHUNK), jnp.float32),
        ] + [pltpu.SemaphoreType.DMA] * NBUF,
        compiler_params=pltpu.CompilerParams(use_tc_tiling_on_sc=True,
                                             needs_layout_passes=False),
    )
    def k(idxt_hbm, wpad_hbm, out_hbm, idx_v, gbufs, tbuf, *gsems):
        nc = plsc.get_sparse_core_info().num_cores
        wid = lax.axis_index("s") * nc + lax.axis_index("c")
        base = wid * b_per_w
        row_ids = [lax.iota(jnp.int32, LANES) + LANES * g
                   for g in range(CHUNK // LANES)]

        for jt in range(n_jt):
            b0 = base + jt * CHUNK
            pltpu.sync_copy(idxt_hbm.at[:, pl.ds(b0, CHUNK)], idx_v)

            for b in range(NBUF):
                pltpu.async_copy(wpad_hbm.at[idx_v.at[b]], gbufs.at[b],
                                 gsems[b])

            def step(st, carry):
                for b in range(NBUF):
                    s = st * NBUF + b
                    pltpu.make_async_copy(wpad_hbm.at[idx_v.at[s]],
                                          gbufs.at[b], gsems[b]).wait()

                    def trans_row(dd, c):
                        col = jnp.full((LANES,), dd, jnp.int32)
                        for g in range(CHUNK // LANES):
                            vec = plsc.load_gather(gbufs.at[b],
                                                   [row_ids[g], col])
                            tbuf[dd, pl.ds(LANES * g, LANES)] = vec
                        return c

                    lax.fori_loop(0, d, trans_row, 0)
                    pltpu.sync_copy(tbuf, out_hbm.at[s, :, pl.ds(b0, CHUNK)])
                    sn = jnp.minimum(s + NBUF, seq - 1)
                    pltpu.async_copy(wpad_hbm.at[idx_v.at[sn]], gbufs.at[b],
                                     gsems[b])
                return carry

            lax.fori_loop(0, seq // NBUF, step, 0)

            for b in range(NBUF):
                pltpu.make_async_copy(wpad_hbm.at[idx_v.at[seq - 1]],
                                      gbufs.at[b], gsems[b]).wait()

    return k


def kernel(token_ids, weight):
    batch, seq = token_ids.shape
    vocab, d = weight.shape
    info = plsc.get_sparse_core_info()
    n_workers = info.num_cores * info.num_subcores
    seq_pad = (seq + 7) // 8 * 8
    dpad = 128
    idxt = jnp.pad(token_ids.T.astype(jnp.int32), ((0, seq_pad - seq), (0, 0)))
    wpad = jnp.pad(weight, ((0, 0), (0, dpad - d)))
    o = _build(batch, seq, seq_pad, d, dpad, n_workers)(idxt, wpad)
    return o.transpose(2, 0, 1)


# speedup vs baseline: 2.3787x; 2.3787x over previous
import functools

import jax
import jax.numpy as jnp
from jax import lax
from jax.experimental import pallas as pl
from jax.experimental.pallas import tpu as pltpu
from jax.experimental.pallas import tpu_sc as plsc

LANES = 16
CHUNK = 128
NBUF = 5


@functools.lru_cache(maxsize=None)
def _build(batch: int, seq: int, seq_pad: int, d: int, dpad: int,
           n_workers: int):
    b_per_w = batch // n_workers
    n_jt = b_per_w // CHUNK
    mesh = plsc.VectorSubcoreMesh(core_axis_name="c", subcore_axis_name="s")

    @functools.partial(
        pl.kernel,
        mesh=mesh,
        out_type=jax.ShapeDtypeStruct((seq, d, batch), jnp.float32),
        scratch_types=[
            pltpu.VMEM((seq_pad, CHUNK), jnp.int32),
            pltpu.VMEM((NBUF, CHUNK, dpad), jnp.float32),
            pltpu.VMEM((d, CHUNK), jnp.float32),
        ] + [pltpu.SemaphoreType.DMA] * NBUF,
        compiler_params=pltpu.CompilerParams(use_tc_tiling_on_sc=True,
                                             needs_layout_passes=False),
    )
    def k(idxt_hbm, wpad_hbm, out_hbm, idx_v, gbufs, tbuf, *gsems):
        nc = plsc.get_sparse_core_info().num_cores
        wid = lax.axis_index("s") * nc + lax.axis_index("c")
        base = wid * b_per_w
        row_ids = [lax.iota(jnp.int32, LANES) + LANES * g
                   for g in range(CHUNK // LANES)]

        for jt in range(n_jt):
            b0 = base + jt * CHUNK
            pltpu.sync_copy(idxt_hbm.at[:, pl.ds(b0, CHUNK)], idx_v)

            for b in range(NBUF):
                pltpu.async_copy(wpad_hbm.at[idx_v.at[b]], gbufs.at[b],
                                 gsems[b])

            def step(st, carry):
                for b in range(NBUF):
                    s = st * NBUF + b
                    pltpu.make_async_copy(wpad_hbm.at[idx_v.at[s]],
                                          gbufs.at[b], gsems[b]).wait()

                    @functools.partial(plsc.parallel_loop, 0, d // 4)
                    def _(dq):
                        for u in range(4):
                            dd = dq * 4 + u
                            col = jnp.full((LANES,), dd, jnp.int32)
                            for g in range(CHUNK // LANES):
                                vec = plsc.load_gather(gbufs.at[b],
                                                       [row_ids[g], col])
                                tbuf[dd, pl.ds(LANES * g, LANES)] = vec
                    pltpu.sync_copy(tbuf, out_hbm.at[s, :, pl.ds(b0, CHUNK)])
                    sn = jnp.minimum(s + NBUF, seq - 1)
                    pltpu.async_copy(wpad_hbm.at[idx_v.at[sn]], gbufs.at[b],
                                     gsems[b])
                return carry

            lax.fori_loop(0, seq // NBUF, step, 0)

            for b in range(NBUF):
                pltpu.make_async_copy(wpad_hbm.at[idx_v.at[seq - 1]],
                                      gbufs.at[b], gsems[b]).wait()

    return k


def kernel(token_ids, weight):
    batch, seq = token_ids.shape
    vocab, d = weight.shape
    info = plsc.get_sparse_core_info()
    n_workers = info.num_cores * info.num_subcores
    seq_pad = (seq + 7) // 8 * 8
    dpad = 128
    idxt = jnp.pad(token_ids.T.astype(jnp.int32), ((0, seq_pad - seq), (0, 0)))
    wpad = jnp.pad(weight, ((0, 0), (0, dpad - d)))
    o = _build(batch, seq, seq_pad, d, dpad, n_workers)(idxt, wpad)
    return o.transpose(2, 0, 1)
